# baseline (speedup 1.0000x reference)
import functools

import jax
import jax.numpy as jnp
from jax import lax
from jax.experimental import pallas as pl
from jax.experimental.pallas import tpu as pltpu
from jax.experimental.pallas import tpu_sc as plsc

VOCAB = 100000
EMBED_DIM = 128
BATCH = 4096
HIST = 50

_INFO = plsc.get_sparse_core_info()
_NC = _INFO.num_cores
_NS = _INFO.num_subcores
_NW = _NC * _NS

_B = BATCH * HIST
_PER_W = _B // _NW
_CHUNK = 128
_NCH = _PER_W // _CHUNK
_NBUF = 5
_LOOK = 3


@functools.partial(
    pl.kernel,
    mesh=plsc.VectorSubcoreMesh(core_axis_name="c", subcore_axis_name="s"),
    out_type=jax.ShapeDtypeStruct((_B, EMBED_DIM), jnp.float32),
    scratch_types=[
        pltpu.VMEM((_NCH, _CHUNK), jnp.int32),
        pltpu.VMEM((_NBUF, _CHUNK, EMBED_DIM), jnp.float32),
        pltpu.SemaphoreType.DMA((_NBUF,)),
        pltpu.SemaphoreType.DMA((_NBUF,)),
    ],
)
def _gather_kernel(table_hbm, idx_hbm, out_hbm, idx_v, rows_v, gsem, ssem):
    wid = lax.axis_index("s") * _NC + lax.axis_index("c")
    base = wid * _PER_W
    pltpu.sync_copy(idx_hbm.at[wid], idx_v)

    def issue_gather(j, b):
        pltpu.async_copy(table_hbm.at[idx_v.at[j]], rows_v.at[b], gsem.at[b])

    def wait_gather(b):
        pltpu.make_async_copy(
            table_hbm.at[idx_v.at[0]], rows_v.at[b], gsem.at[b]
        ).wait()

    def issue_store(j, b):
        pltpu.async_copy(
            rows_v.at[b], out_hbm.at[pl.ds(base + j * _CHUNK, _CHUNK)], ssem.at[b]
        )

    def wait_store(b):
        pltpu.make_async_copy(
            rows_v.at[b], out_hbm.at[pl.ds(base, _CHUNK)], ssem.at[b]
        ).wait()

    def step(j, b, do_wait_store, do_issue_gather):
        wait_gather(b)
        issue_store(j, b)
        if do_issue_gather:
            bn = (b + _LOOK) % _NBUF
            if do_wait_store:
                wait_store(bn)
            issue_gather(j + _LOOK, bn)

    for jp in range(_LOOK):
        issue_gather(jp, jp)

    for b in range(_NBUF):
        step(b, b, b >= _NBUF - _LOOK, True)

    def body(g, carry):
        j0 = g * _NBUF
        for b in range(_NBUF):
            step(j0 + b, b, True, True)
        return carry

    lax.fori_loop(1, _NCH // _NBUF - 1, body, 0)

    for b in range(_NBUF):
        j = _NCH - _NBUF + b
        step(j, b, True, j + _LOOK < _NCH)

    for j in range(_NCH - _NBUF, _NCH):
        wait_store(j % _NBUF)


def kernel(token_ids, table):
    idx = token_ids.reshape(_NW, _NCH, _CHUNK).astype(jnp.int32)
    out = _gather_kernel(table, idx)
    return out.reshape(BATCH, HIST, EMBED_DIM)


# speedup vs baseline: 1.0015x; 1.0015x over previous
import functools

import jax
import jax.numpy as jnp
from jax import lax
from jax.experimental import pallas as pl
from jax.experimental.pallas import tpu as pltpu
from jax.experimental.pallas import tpu_sc as plsc

VOCAB = 100000
EMBED_DIM = 128
BATCH = 4096
HIST = 50

_INFO = plsc.get_sparse_core_info()
_NC = _INFO.num_cores
_NS = _INFO.num_subcores
_NW = _NC * _NS

_B = BATCH * HIST
_PER_W = _B // _NW
_CHUNK = 128
_NCH = _PER_W // _CHUNK
_NBUF = 5
_LOOK = 2


@functools.partial(
    pl.kernel,
    mesh=plsc.VectorSubcoreMesh(core_axis_name="c", subcore_axis_name="s"),
    out_type=jax.ShapeDtypeStruct((_B, EMBED_DIM), jnp.float32),
    scratch_types=[
        pltpu.VMEM((_NCH, _CHUNK), jnp.int32),
        pltpu.VMEM((_NBUF, _CHUNK, EMBED_DIM), jnp.float32),
        pltpu.SemaphoreType.DMA((_NBUF,)),
        pltpu.SemaphoreType.DMA((_NBUF,)),
    ],
)
def _gather_kernel(table_hbm, idx_hbm, out_hbm, idx_v, rows_v, gsem, ssem):
    wid = lax.axis_index("s") * _NC + lax.axis_index("c")
    base = wid * _PER_W
    pltpu.sync_copy(idx_hbm.at[wid], idx_v)

    def issue_gather(j, b):
        pltpu.async_copy(table_hbm.at[idx_v.at[j]], rows_v.at[b], gsem.at[b])

    def wait_gather(b):
        pltpu.make_async_copy(
            table_hbm.at[idx_v.at[0]], rows_v.at[b], gsem.at[b]
        ).wait()

    def issue_store(j, b):
        pltpu.async_copy(
            rows_v.at[b], out_hbm.at[pl.ds(base + j * _CHUNK, _CHUNK)], ssem.at[b]
        )

    def wait_store(b):
        pltpu.make_async_copy(
            rows_v.at[b], out_hbm.at[pl.ds(base, _CHUNK)], ssem.at[b]
        ).wait()

    def step(j, b, do_wait_store, do_issue_gather):
        wait_gather(b)
        issue_store(j, b)
        if do_issue_gather:
            bn = (b + _LOOK) % _NBUF
            if do_wait_store:
                wait_store(bn)
            issue_gather(j + _LOOK, bn)

    for jp in range(_LOOK):
        issue_gather(jp, jp)

    for b in range(_NBUF):
        step(b, b, b >= _NBUF - _LOOK, True)

    def body(g, carry):
        j0 = g * _NBUF
        for b in range(_NBUF):
            step(j0 + b, b, True, True)
        return carry

    lax.fori_loop(1, _NCH // _NBUF - 1, body, 0)

    for b in range(_NBUF):
        j = _NCH - _NBUF + b
        step(j, b, True, j + _LOOK < _NCH)

    for j in range(_NCH - _NBUF, _NCH):
        wait_store(j % _NBUF)


def kernel(token_ids, table):
    idx = token_ids.reshape(_NW, _NCH, _CHUNK).astype(jnp.int32)
    out = _gather_kernel(table, idx)
    return out.reshape(BATCH, HIST, EMBED_DIM)
